# Initial kernel scaffold; baseline (speedup 1.0000x reference)
import jax
import jax.numpy as jnp
from jax.experimental import pallas as pl


def kernel(token_ids, table):
    raise NotImplementedError("write your pallas kernel here")


# speedup vs baseline: 3.9488x; 3.9488x over previous
import jax
import jax.numpy as jnp
from jax import lax
from jax.experimental import pallas as pl
from jax.experimental.pallas import tpu as pltpu
from jax.experimental.pallas import tpu_sc as plsc

D = 64
B_TOK = 4096 * 200
IDX_MINOR = 128
N_IDX_ROWS = B_TOK // IDX_MINOR
NC, NS = 2, 16
NW = NC * NS
ROWS_PER_W = N_IDX_ROWS // NW
J = 4
CHUNK = J * IDX_MINOR
N_CHUNKS = ROWS_PER_W // J


def _emb_body(idx_hbm, table_hbm, out_hbm, idx_v, rows_v, sem):
    wid = lax.axis_index("s") * NC + lax.axis_index("c")
    base_row = wid * ROWS_PER_W

    def body(g, carry):
        r0 = base_row + g * J
        pltpu.sync_copy(idx_hbm.at[pl.ds(r0, J)], idx_v)
        copies = [
            pltpu.async_copy(
                table_hbm.at[idx_v.at[j]],
                rows_v.at[pl.ds(j * IDX_MINOR, IDX_MINOR)],
                sem,
            )
            for j in range(J)
        ]
        for cp in copies:
            cp.wait()
        pltpu.sync_copy(rows_v, out_hbm.at[pl.ds(r0 * IDX_MINOR, CHUNK)])
        return carry

    lax.fori_loop(0, N_CHUNKS, body, 0)


def kernel(token_ids, table):
    idx = token_ids.reshape(N_IDX_ROWS, IDX_MINOR).astype(jnp.int32)
    mesh = plsc.VectorSubcoreMesh(core_axis_name="c", subcore_axis_name="s")
    out = pl.kernel(
        _emb_body,
        out_type=jax.ShapeDtypeStruct((B_TOK, D), jnp.float32),
        mesh=mesh,
        compiler_params=pltpu.CompilerParams(use_tc_tiling_on_sc=False),
        scratch_types=[
            pltpu.VMEM((J, IDX_MINOR), jnp.int32),
            pltpu.VMEM((CHUNK, D), jnp.float32),
            pltpu.SemaphoreType.DMA,
        ],
    )(idx, table)
    return out.reshape(token_ids.shape[0], token_ids.shape[1], D)


# speedup vs baseline: 4.2631x; 1.0796x over previous
import jax
import jax.numpy as jnp
from jax import lax
from jax.experimental import pallas as pl
from jax.experimental.pallas import tpu as pltpu
from jax.experimental.pallas import tpu_sc as plsc

D = 64
B_TOK = 4096 * 200
IDX_MINOR = 128
N_IDX_ROWS = B_TOK // IDX_MINOR
NC, NS = 2, 16
NW = NC * NS
ROWS_PER_W = N_IDX_ROWS // NW
J = 4
CHUNK = J * IDX_MINOR
N_CHUNKS = ROWS_PER_W // J


def _emb_body(idx_hbm, table_hbm, out_hbm, idx_v, rows_v, sem0, sem1):
    wid = lax.axis_index("s") * NC + lax.axis_index("c")
    base_row = wid * ROWS_PER_W
    sems = (sem0, sem1)

    pltpu.sync_copy(idx_hbm.at[pl.ds(base_row, ROWS_PER_W)], idx_v)

    def fire(g, b):
        for j in range(J):
            pltpu.async_copy(
                table_hbm.at[idx_v.at[g * J + j]],
                rows_v.at[b, pl.ds(j * IDX_MINOR, IDX_MINOR)],
                sems[b],
            )

    def drain(b):
        pltpu.make_async_copy(
            out_hbm.at[pl.ds(0, CHUNK)], rows_v.at[b], sems[b]
        ).wait()

    def writeback(g, b):
        pltpu.sync_copy(
            rows_v.at[b],
            out_hbm.at[pl.ds((base_row + g * J) * IDX_MINOR, CHUNK)],
        )

    fire(0, 0)

    def body(k, carry):
        g0 = 2 * k
        fire(g0 + 1, 1)
        drain(0)
        writeback(g0, 0)
        fire(g0 + 2, 0)
        drain(1)
        writeback(g0 + 1, 1)
        return carry

    lax.fori_loop(0, N_CHUNKS // 2 - 1, body, 0)

    fire(N_CHUNKS - 1, 1)
    drain(0)
    writeback(N_CHUNKS - 2, 0)
    drain(1)
    writeback(N_CHUNKS - 1, 1)


def kernel(token_ids, table):
    idx = token_ids.reshape(N_IDX_ROWS, IDX_MINOR).astype(jnp.int32)
    mesh = plsc.VectorSubcoreMesh(core_axis_name="c", subcore_axis_name="s")
    out = pl.kernel(
        _emb_body,
        out_type=jax.ShapeDtypeStruct((B_TOK, D), jnp.float32),
        mesh=mesh,
        compiler_params=pltpu.CompilerParams(use_tc_tiling_on_sc=False),
        scratch_types=[
            pltpu.VMEM((ROWS_PER_W, IDX_MINOR), jnp.int32),
            pltpu.VMEM((2, CHUNK, D), jnp.float32),
            pltpu.SemaphoreType.DMA,
            pltpu.SemaphoreType.DMA,
        ],
    )(idx, table)
    return out.reshape(token_ids.shape[0], token_ids.shape[1], D)
